# Initial kernel scaffold; baseline (speedup 1.0000x reference)
import jax
import jax.numpy as jnp
from jax.experimental import pallas as pl


def kernel(pred_boxes, pred_classes, pred_keypoints, pred_depths, true_boxes, true_keypoints, true_depths, anchors, true_labels):
    raise NotImplementedError("write your pallas kernel here")


# speedup vs baseline: 3.1027x; 3.1027x over previous
import jax
import jax.numpy as jnp
from jax import lax
from jax.experimental import pallas as pl

_B, _A, _O = 16, 16384, 32
_A_BLK = 2048
_NBLK = _A // _A_BLK
_NEG_ROWS = _B * _A // 128


def _match_body(tb_ref, tl_ref, an_ref, pb_ref, pc_ref, negv_ref, stats_ref):
    @pl.when((pl.program_id(0) == 0) & (pl.program_id(1) == 0))
    def _init():
        stats_ref[...] = jnp.zeros_like(stats_ref)

    an = an_ref[...]
    acx, acy, aw, ah = an[0:1], an[1:2], an[2:3], an[3:4]
    ax1 = acx - 0.5 * aw
    ay1 = acy - 0.5 * ah
    ax2 = acx + 0.5 * aw
    ay2 = acy + 0.5 * ah
    area_a = aw * ah

    tb = tb_ref[0]
    tx1, ty1, tx2, ty2 = tb[:, 0:1], tb[:, 1:2], tb[:, 2:3], tb[:, 3:4]
    lbl = tl_ref[0]
    valid = lbl >= 0.0
    area_t = (tx2 - tx1) * (ty2 - ty1)

    wx = jnp.maximum(jnp.minimum(ax2, tx2) - jnp.maximum(ax1, tx1), 0.0)
    wy = jnp.maximum(jnp.minimum(ay2, ty2) - jnp.maximum(ay1, ty1), 0.0)
    inter = wx * wy
    union = (area_a + area_t) - inter
    iou = inter / union
    iou = jnp.where(valid, iou, -1.0)

    mx = jnp.max(iou, axis=0, keepdims=True)
    pos = (jnp.abs(mx - iou) < 1e-6) & (iou > 0.5)
    posf = pos.astype(jnp.float32)
    negm = mx < 0.5

    n_pos_s = jnp.sum(posf)
    n_neg_s = jnp.sum(negm.astype(jnp.float32))

    pb = pb_ref[0]
    inv_w = 10.0 / aw
    inv_h = 10.0 / ah
    tcx = 0.5 * (tx1 + tx2)
    tcy = 0.5 * (ty1 + ty2)
    log_tw = jnp.log(tx2 - tx1)
    log_th = jnp.log(ty2 - ty1)
    log_aw = jnp.log(aw)
    log_ah = jnp.log(ah)
    g0 = (tcx - acx) * inv_w
    g1 = (tcy - acy) * inv_h
    g2 = (log_tw - log_aw) * 5.0
    g3 = (log_th - log_ah) * 5.0
    okw = jnp.logical_not(jnp.isnan(log_tw))
    okh = jnp.logical_not(jnp.isnan(log_th))

    def sl1(p_row, g, m):
        d = jnp.abs(p_row - g)
        v = jnp.where(d < 1.0, 0.5 * d * d, d - 0.5)
        return jnp.sum(jnp.where(m, v, 0.0))

    lb_s = (sl1(pb[0:1], g0, pos) + sl1(pb[1:2], g1, pos)
            + sl1(pb[2:3], g2, pos & okw) + sl1(pb[3:4], g3, pos & okh))

    pc = pc_ref[0]
    l0, l1 = pc[0:1], pc[1:2]
    mmx = jnp.maximum(l0, l1)
    lse = mmx + jnp.log(jnp.exp(l0 - mmx) + jnp.exp(l1 - mmx))
    ls0 = l0 - lse
    ls1 = l1 - lse
    ce_neg = -ls0
    is1 = lbl == 1.0
    w_bo = jnp.where(is1, 4.0, 1.0)
    ce_pos = jnp.where(is1, -ls1, -ls0)
    s_pos_s = jnp.sum(jnp.where(pos, w_bo * ce_pos, 0.0))
    w_pos_s = jnp.sum(posf * w_bo)

    negv_ref[0] = jnp.where(negm, ce_neg, -1.0)

    cols = lax.broadcasted_iota(jnp.int32, (1, 128), 1)
    upd = (jnp.where(cols == 0, n_pos_s, 0.0)
           + jnp.where(cols == 1, n_neg_s, 0.0)
           + jnp.where(cols == 2, lb_s, 0.0)
           + jnp.where(cols == 3, s_pos_s, 0.0)
           + jnp.where(cols == 4, w_pos_s, 0.0))
    stats_ref[...] += upd


def _select_body(negv_ref, stats_ref, out_ref):
    st = stats_ref[...]
    n_pos = st[0, 0]
    n_neg = st[0, 1]
    lb = st[0, 2]
    s_pos = st[0, 3]
    w_pos_sum = st[0, 4]

    k_i = jnp.minimum(n_pos.astype(jnp.int32) * 10, n_neg.astype(jnp.int32))
    k = k_i.astype(jnp.float32)

    def body(_, carry):
        lo, hi = carry
        mid = lo + (hi - lo) // 2
        t = lax.bitcast_convert_type(mid, jnp.float32)
        c = jnp.sum((negv_ref[...] >= t).astype(jnp.float32))
        ge_k = c >= k
        return jnp.where(ge_k, mid, lo), jnp.where(ge_k, hi, mid)

    lo, _ = lax.fori_loop(0, 31, body, (jnp.int32(0), jnp.int32(0x7F800000)))
    t = lax.bitcast_convert_type(lo, jnp.float32)
    v = negv_ref[...]
    gt = v > t
    sum_gt = jnp.sum(jnp.where(gt, v, 0.0))
    cnt_gt = jnp.sum(gt.astype(jnp.float32))
    s_neg = jnp.where(k > 0.0, sum_gt + (k - cnt_gt) * t, 0.0)

    n_pos_c = jnp.maximum(n_pos, 1.0)
    lb_c = jnp.where(jnp.isfinite(lb), lb, 0.0)
    loss_boxes = lb_c / n_pos_c
    denom = w_pos_sum + k
    lc = (s_pos + s_neg) / denom
    lc = jnp.where(jnp.isfinite(lc), lc, 0.0)
    loss_classes = 10.0 * lc / n_pos_c
    total = loss_boxes + loss_classes

    cols = lax.broadcasted_iota(jnp.int32, (1, 128), 1)
    out_ref[...] = (jnp.where(cols == 0, loss_boxes, 0.0)
                    + jnp.where(cols == 1, loss_classes, 0.0)
                    + jnp.where(cols == 2, total, 0.0))


def _match_call(tb, tl, an_t, pb_t, pc_t):
    return pl.pallas_call(
        _match_body,
        grid=(_B, _NBLK),
        in_specs=[
            pl.BlockSpec((1, _O, 4), lambda i, j: (i, 0, 0)),
            pl.BlockSpec((1, _O, 1), lambda i, j: (i, 0, 0)),
            pl.BlockSpec((4, _A_BLK), lambda i, j: (0, j)),
            pl.BlockSpec((1, 4, _A_BLK), lambda i, j: (i, 0, j)),
            pl.BlockSpec((1, 2, _A_BLK), lambda i, j: (i, 0, j)),
        ],
        out_specs=[
            pl.BlockSpec((1, 1, _A_BLK), lambda i, j: (i, 0, j)),
            pl.BlockSpec((1, 128), lambda i, j: (0, 0)),
        ],
        out_shape=[
            jax.ShapeDtypeStruct((_B, 1, _A), jnp.float32),
            jax.ShapeDtypeStruct((1, 128), jnp.float32),
        ],
    )(tb, tl, an_t, pb_t, pc_t)


def _select_call(negv, stats):
    return pl.pallas_call(
        _select_body,
        grid=(1,),
        in_specs=[
            pl.BlockSpec((_NEG_ROWS, 128), lambda i: (0, 0)),
            pl.BlockSpec((1, 128), lambda i: (0, 0)),
        ],
        out_specs=pl.BlockSpec((1, 128), lambda i: (0, 0)),
        out_shape=jax.ShapeDtypeStruct((1, 128), jnp.float32),
    )(negv, stats)


def kernel(pred_boxes, pred_classes, pred_keypoints, pred_depths, true_boxes,
           true_keypoints, true_depths, anchors, true_labels):
    pb_t = jnp.transpose(pred_boxes, (0, 2, 1))
    pc_t = jnp.transpose(pred_classes, (0, 2, 1))
    an_t = jnp.transpose(anchors, (1, 0))
    tl_f = true_labels.astype(jnp.float32)
    negv, stats = _match_call(true_boxes, tl_f, an_t, pb_t, pc_t)
    out = _select_call(negv.reshape(_NEG_ROWS, 128), stats)
    loss_boxes = out[0, 0].reshape(())
    loss_classes = out[0, 1].reshape(())
    total = out[0, 2].reshape(())
    return (loss_boxes, loss_classes, total)
